# Initial kernel scaffold; baseline (speedup 1.0000x reference)
import jax
import jax.numpy as jnp
from jax.experimental import pallas as pl

N = 10000; E = 320000; Fd = 128; H = 4; C = 32; G = 16


def _gatv2(x, src, dst, Wl, Wr, att, bias):
    xl = (x @ Wl).reshape(N, H, C)
    xr = (x @ Wr).reshape(N, H, C)
    e = jax.nn.leaky_relu(xl[src] + xr[dst], negative_slope=0.2)
    logits = jnp.sum(e * att[None, :, :], axis=-1)
    lmax = jax.ops.segment_max(logits, dst, num_segments=N)
    lmax = jnp.where(jnp.isfinite(lmax), lmax, 0.0)
    ex = jnp.exp(logits - lmax[dst])
    den = jax.ops.segment_sum(ex, dst, num_segments=N)
    alpha = ex / (den[dst] + 1e-16)
    out = jax.ops.segment_sum(alpha[:, :, None] * xl[src], dst, num_segments=N)
    return out.reshape(N, H * C) + bias


def _graph_norm(x, batch, w, b, ms):
    cnt = jax.ops.segment_sum(jnp.ones((N,), jnp.float32), batch, num_segments=G)[:, None]
    mean = jax.ops.segment_sum(x, batch, num_segments=G) / jnp.maximum(cnt, 1.0)
    sub = x - ms * mean[batch]
    var = jax.ops.segment_sum(sub * sub, batch, num_segments=G) / jnp.maximum(cnt, 1.0)
    return w * sub / jnp.sqrt(var[batch] + 1e-5) + b


def _attn_aggr(x, batch, g1w, g1b, g2w, g2b):
    gate = jax.nn.relu(x @ g1w + g1b) @ g2w + g2b
    gmax = jax.ops.segment_max(gate, batch, num_segments=G)
    gmax = jnp.where(jnp.isfinite(gmax), gmax, 0.0)
    eg = jnp.exp(gate - gmax[batch])
    den = jax.ops.segment_sum(eg, batch, num_segments=G)
    alpha = eg / (den[batch] + 1e-16)
    return jax.ops.segment_sum(alpha * x, batch, num_segments=G)


def _ident_body(x_ref, o_ref):
    o_ref[...] = x_ref[...]


def kernel(x, edge_index, batch, Wl1, Wr1, att1, bias1, gnw1, gnb1, gnm1, Wl2, Wr2, att2, bias2, gnw2, gnb2, gnm2, Wl3, Wr3, att3, bias3, gnw3, gnb3, gnm3, g1w, g1b, g2w, g2b):
    src, dst = edge_index[0], edge_index[1]
    h = x
    for (Wl, Wr, att, bias, gw, gb, gm) in (
        (Wl1, Wr1, att1, bias1, gnw1, gnb1, gnm1),
        (Wl2, Wr2, att2, bias2, gnw2, gnb2, gnm2),
        (Wl3, Wr3, att3, bias3, gnw3, gnb3, gnm3)):
        h = _gatv2(h, src, dst, Wl, Wr, att, bias)
        h = _graph_norm(h, batch, gw, gb, gm)
        h = jax.nn.relu(h)
    out = _attn_aggr(h, batch, g1w, g1b, g2w, g2b)
    return pl.pallas_call(
        _ident_body,
        out_shape=jax.ShapeDtypeStruct(out.shape, out.dtype),
    )(out)


# speedup vs baseline: 35.5905x; 35.5905x over previous
import functools

import jax
import jax.numpy as jnp
from jax import lax
from jax.experimental import pallas as pl
from jax.experimental.pallas import tpu as pltpu
from jax.experimental.pallas import tpu_sc as plsc

N = 10000
E = 320000
Fd = 128
H = 4
C = 32
HC = 128
G = 16

_NC = 2
_NS = 16
_NW = _NC * _NS
_EPW = E // _NW
_K = 80
_NCHUNK = _EPW // _K
_W = 144
_RPT = N // _NS


_sc_mesh = plsc.VectorSubcoreMesh(core_axis_name="c", subcore_axis_name="s")


@functools.partial(
    pl.kernel,
    mesh=_sc_mesh,
    compiler_params=pltpu.CompilerParams(use_tc_tiling_on_sc=False,
                                         needs_layout_passes=False),
    out_type=jax.ShapeDtypeStruct((_NC, N, _W), jnp.float32),
    scratch_types=[
        pltpu.VMEM((_K,), jnp.int32),
        pltpu.VMEM((_K,), jnp.int32),
        pltpu.VMEM((_K, HC), jnp.float32),
        pltpu.VMEM((_K, HC), jnp.float32),
        pltpu.VMEM((_K, _W), jnp.float32),
        pltpu.VMEM((HC,), jnp.float32),
        pltpu.VMEM_SHARED((N, _W), jnp.float32),
        pltpu.SemaphoreType.DMA,
        pltpu.SemaphoreType.DMA,
    ],
)
def _edge_kernel(xl_hbm, xr_hbm, src_hbm, dst_hbm, att_hbm, out_hbm,
                 src_v, dst_v, xl_v, xr_v, y_v, att_v, acc_sh, sem1, sem2):
    c = lax.axis_index("c")
    s = lax.axis_index("s")
    wid = s * _NC + c

    zero = jnp.zeros((16,), jnp.float32)

    def _zrow(r, _):
        for j in range(_W // 16):
            y_v[r, pl.ds(16 * j, 16)] = zero
        return 0
    lax.fori_loop(0, _K, _zrow, 0)

    def _zacc(i, _):
        pltpu.sync_copy(y_v.at[pl.ds(0, 25)],
                        acc_sh.at[pl.ds(s * _RPT + i * 25, 25)])
        return 0
    lax.fori_loop(0, _RPT // 25, _zacc, 0)

    pltpu.sync_copy(att_hbm, att_v)
    plsc.subcore_barrier()

    attj = [att_v[pl.ds(16 * j, 16)] for j in range(8)]
    li = lax.iota(jnp.int32, 16)
    hm = [li == h for h in range(H)]

    def _edge(e, _):
        avals = []
        ns = [None] * H
        for j in range(8):
            a = xl_v[e, pl.ds(16 * j, 16)]
            b = xr_v[e, pl.ds(16 * j, 16)]
            sab = a + b
            t = jnp.maximum(sab, sab * 0.2)
            m = t * attj[j]
            h = j // 2
            ns[h] = m if ns[h] is None else ns[h] + m
            avals.append(a)
        exs = []
        for h in range(H):
            l = jnp.sum(ns[h])
            exs.append(jnp.exp(lax.broadcast(l, (16,))))
        for j in range(8):
            y_v[e, pl.ds(16 * j, 16)] = avals[j] * exs[j // 2]
        exlane = jnp.where(hm[0], exs[0],
                  jnp.where(hm[1], exs[1],
                   jnp.where(hm[2], exs[2],
                    jnp.where(hm[3], exs[3], zero))))
        y_v[e, pl.ds(HC, 16)] = exlane
        return 0

    def _chunk(t, _):
        base = wid * _EPW + t * _K
        pltpu.sync_copy(src_hbm.at[pl.ds(base, _K)], src_v)
        pltpu.sync_copy(dst_hbm.at[pl.ds(base, _K)], dst_v)
        cp1 = pltpu.async_copy(xl_hbm.at[src_v], xl_v, sem1)
        cp2 = pltpu.async_copy(xr_hbm.at[dst_v], xr_v, sem2)
        cp1.wait()
        cp2.wait()
        lax.fori_loop(0, _K, _edge, 0)
        pltpu.sync_copy(y_v, acc_sh.at[dst_v], add=True)
        return 0

    lax.fori_loop(0, _NCHUNK, _chunk, 0)

    plsc.subcore_barrier()
    pltpu.sync_copy(acc_sh.at[pl.ds(s * _RPT, _RPT)],
                    out_hbm.at[c, pl.ds(s * _RPT, _RPT)])


def _project_body(h_ref, wl_ref, wr_ref, xl_ref, xr_ref):
    h = h_ref[...]
    xl_ref[...] = jnp.dot(h, wl_ref[...], preferred_element_type=jnp.float32)
    xr_ref[...] = jnp.dot(h, wr_ref[...], preferred_element_type=jnp.float32)


def _project(h, Wl, Wr):
    return pl.pallas_call(
        _project_body,
        out_shape=(jax.ShapeDtypeStruct((N, HC), jnp.float32),
                   jax.ShapeDtypeStruct((N, HC), jnp.float32)),
    )(h, Wl, Wr)


def _merge_body(acc0_ref, acc1_ref, bias_ref, hgat_ref):
    acc0 = acc0_ref[...]
    acc1 = acc1_ref[...]
    num = acc0[:, :HC] + acc1[:, :HC]
    den = acc0[:, HC:HC + H] + acc1[:, HC:HC + H]
    dex = jnp.concatenate(
        [jnp.broadcast_to(den[:, h:h + 1], (N, C)) for h in range(H)], axis=1)
    hgat_ref[...] = num / (dex + 1e-16) + bias_ref[...]


def _merge(acc0, acc1, bias):
    return pl.pallas_call(
        _merge_body,
        out_shape=jax.ShapeDtypeStruct((N, HC), jnp.float32),
    )(acc0, acc1, bias)


def _segment_mats(brow, bcol):
    iota_c = lax.broadcasted_iota(jnp.int32, (G, 1), 0)
    ohT = (iota_c == brow[0:1, :]).astype(jnp.float32)
    iota_r = lax.broadcasted_iota(jnp.int32, (1, G), 1)
    oh = (bcol == iota_r).astype(jnp.float32)
    return ohT, oh


def _graph_norm(hgat, ohT, oh, gw, gb, gm):
    hp = jax.lax.Precision.HIGHEST
    cnt = jnp.maximum(jnp.sum(ohT, axis=1, keepdims=True), 1.0)
    mean = jnp.dot(ohT, hgat, precision=hp, preferred_element_type=jnp.float32) / cnt
    meanb = jnp.dot(oh, mean, precision=hp, preferred_element_type=jnp.float32)
    sub = hgat - gm * meanb
    var = jnp.dot(ohT, sub * sub, precision=hp, preferred_element_type=jnp.float32) / cnt
    varb = jnp.dot(oh, var, precision=hp, preferred_element_type=jnp.float32)
    return gw * sub / jnp.sqrt(varb + 1e-5) + gb


def _norm_project_body(hgat_ref, gw_ref, gb_ref, gm_ref,
                       brow_ref, bcol_ref, wl_ref, wr_ref, xl_ref, xr_ref):
    hgat = hgat_ref[...]
    ohT, oh = _segment_mats(brow_ref[...], bcol_ref[...])
    hn = _graph_norm(hgat, ohT, oh, gw_ref[...], gb_ref[...], gm_ref[...])
    h2 = jnp.maximum(hn, 0.0)
    xl_ref[...] = jnp.dot(h2, wl_ref[...], preferred_element_type=jnp.float32)
    xr_ref[...] = jnp.dot(h2, wr_ref[...], preferred_element_type=jnp.float32)


def _norm_project(hgat, gw, gb, gm, brow, bcol, Wl, Wr):
    return pl.pallas_call(
        _norm_project_body,
        out_shape=(jax.ShapeDtypeStruct((N, HC), jnp.float32),
                   jax.ShapeDtypeStruct((N, HC), jnp.float32)),
        compiler_params=pltpu.CompilerParams(vmem_limit_bytes=60 * 1024 * 1024),
    )(hgat, gw, gb, gm, brow, bcol, Wl, Wr)


def _final_body(hgat_ref, gw_ref, gb_ref, gm_ref,
                brow_ref, bcol_ref, g1w_ref, g1b_ref, g2w_ref, g2b_ref, out_ref):
    hgat = hgat_ref[...]
    ohT, oh = _segment_mats(brow_ref[...], bcol_ref[...])
    hn = _graph_norm(hgat, ohT, oh, gw_ref[...], gb_ref[...], gm_ref[...])
    h2 = jnp.maximum(hn, 0.0)
    g1 = jnp.maximum(
        jnp.dot(h2, g1w_ref[...], preferred_element_type=jnp.float32)
        + g1b_ref[...], 0.0)
    gate = jnp.dot(g1, g2w_ref[...], preferred_element_type=jnp.float32) \
        + g2b_ref[...]
    hp = jax.lax.Precision.HIGHEST
    eg = jnp.exp(gate)
    den = jnp.dot(ohT, eg, precision=hp, preferred_element_type=jnp.float32)
    denb = jnp.dot(oh, den, precision=hp, preferred_element_type=jnp.float32)
    alpha = eg / (denb + 1e-16)
    out_ref[...] = jnp.dot(ohT, alpha * h2, precision=hp,
                           preferred_element_type=jnp.float32)


def _final(hgat, gw, gb, gm, brow, bcol, g1w, g1b, g2w, g2b):
    return pl.pallas_call(
        _final_body,
        out_shape=jax.ShapeDtypeStruct((G, HC), jnp.float32),
        compiler_params=pltpu.CompilerParams(vmem_limit_bytes=60 * 1024 * 1024),
    )(hgat, gw, gb, gm, brow, bcol, g1w, g1b, g2w, g2b)


def kernel(x, edge_index, batch, Wl1, Wr1, att1, bias1, gnw1, gnb1, gnm1,
           Wl2, Wr2, att2, bias2, gnw2, gnb2, gnm2,
           Wl3, Wr3, att3, bias3, gnw3, gnb3, gnm3, g1w, g1b, g2w, g2b):
    src = edge_index[0]
    dst = edge_index[1]
    brow = jnp.broadcast_to(batch[None, :], (8, N))
    bcol = batch[:, None]

    layers = (
        (Wl1, Wr1, att1, bias1, gnw1, gnb1, gnm1),
        (Wl2, Wr2, att2, bias2, gnw2, gnb2, gnm2),
        (Wl3, Wr3, att3, bias3, gnw3, gnb3, gnm3),
    )

    xl, xr = _project(x, Wl1, Wr1)
    out = None
    for i, (Wl, Wr, att, bias, gw, gb, gm) in enumerate(layers):
        accs = _edge_kernel(xl, xr, src, dst, att.reshape(-1))
        b2 = bias.reshape(1, HC)
        gw2, gb2, gm2 = gw.reshape(1, HC), gb.reshape(1, HC), gm.reshape(1, HC)
        hgat = _merge(accs[0], accs[1], b2)
        if i < 2:
            nWl, nWr = layers[i + 1][0], layers[i + 1][1]
            xl, xr = _norm_project(hgat, gw2, gb2, gm2, brow, bcol, nWl, nWr)
        else:
            out = _final(hgat, gw2, gb2, gm2, brow, bcol,
                         g1w, g1b.reshape(1, 128), g2w, g2b.reshape(1, 1))
    return out
